# baseline (speedup 1.0000x reference)
import functools

import jax
import jax.numpy as jnp
from jax import lax
from jax.experimental import pallas as pl
from jax.experimental.pallas import tpu as pltpu
from jax.experimental.pallas import tpu_sc as plsc

N = 10000
E = 320000
DF = 128
ED = 32
HD = 200
K = 15
KP = 16

NC = 2
NS = 16
NW = NC * NS
EPW = E // NW
CHUNK = 120
EPP = 10080
NCHUNK = EPP // CHUNK
NBUF = 4
NGROUP = NCHUNK // NBUF
OW = 128
NP = 10240
RPT = NP // NS

def _dot(a, b):
    return jnp.dot(a, b, preferred_element_type=jnp.float32)


def _dot_t(a, b):
    return lax.dot_general(a, b, (((0,), (0,)), ((), ())),
                           preferred_element_type=jnp.float32)


@functools.lru_cache(maxsize=None)
def _make_seg_sum(width, with_deg):
    mesh = plsc.VectorSubcoreMesh(core_axis_name="c", subcore_axis_name="s",
                                  num_cores=NC, num_subcores=NS)
    out_type = [jax.ShapeDtypeStruct((NC, N, OW), jnp.float32)]
    nsem = 3 * NBUF if with_deg else 2 * NBUF
    scratch = [
        pltpu.VMEM((NCHUNK, CHUNK), jnp.int32),
        pltpu.VMEM((NCHUNK, CHUNK), jnp.int32),
        pltpu.VMEM((NBUF, CHUNK, width), jnp.float32),
        pltpu.VMEM((RPT, width), jnp.float32),
        pltpu.VMEM_SHARED((NP, width), jnp.float32),
    ] + [pltpu.SemaphoreType.DMA] * nsem
    if with_deg:
        scratch += [
            pltpu.VMEM((CHUNK, 16), jnp.float32),
            pltpu.VMEM((RPT, 16), jnp.float32),
            pltpu.VMEM_SHARED((NP, 16), jnp.float32),
        ]

    def body(table_hbm, src_hbm, dst_hbm, *refs):
        if with_deg:
            (out_hbm, vsrc, vdst, vrows, vstage, acc) = refs[:6]
            sems = refs[6:6 + nsem]
            vones, vstaged, accd = refs[6 + nsem:]
            gsem, ssem, osem = (sems[:NBUF], sems[NBUF:2 * NBUF],
                                sems[2 * NBUF:])
        else:
            (out_hbm, vsrc, vdst, vrows, vstage, acc) = refs[:6]
            sems = refs[6:6 + nsem]
            gsem, ssem = sems[:NBUF], sems[NBUF:]
        c = lax.axis_index("c")
        s = lax.axis_index("s")
        wid = c * NS + s
        zeros16 = jnp.zeros((16,), jnp.float32)

        def zrow(i, _):
            for j0 in range(width // 16):
                vstage[i, pl.ds(j0 * 16, 16)] = zeros16
            if with_deg:
                vstaged[i, pl.ds(0, 16)] = zeros16
            return 0
        lax.fori_loop(0, RPT, zrow, 0)
        pltpu.sync_copy(vstage, acc.at[pl.ds(s * RPT, RPT)])
        if with_deg:
            pltpu.sync_copy(vstaged, accd.at[pl.ds(s * RPT, RPT)])
            ones16 = jnp.ones((16,), jnp.float32)

            def orow(i, _):
                vones[i, pl.ds(0, 16)] = ones16
                return 0
            lax.fori_loop(0, CHUNK, orow, 0)
        plsc.subcore_barrier()

        pltpu.sync_copy(src_hbm.at[wid], vsrc)
        pltpu.sync_copy(dst_hbm.at[wid], vdst)

        factor = OW // width

        def srow(i, _):
            for k in range(CHUNK // 16):
                vsrc[i, pl.ds(k * 16, 16)] = vsrc[i, pl.ds(k * 16, 16)] * factor
            return 0
        lax.fori_loop(0, NCHUNK, srow, 0)

        def fire_gather(j, b):
            pltpu.async_copy(table_hbm.at[vsrc.at[j]], vrows.at[b], gsem[b])

        def wait_gather(j, b):
            pltpu.make_async_copy(table_hbm.at[vsrc.at[j]], vrows.at[b],
                                  gsem[b]).wait()

        def fire_scatter(j, b):
            pltpu.async_copy(vrows.at[b], acc.at[vdst.at[j]], ssem[b],
                             add=True)
            if with_deg:
                pltpu.async_copy(vones, accd.at[vdst.at[j]], osem[b],
                                 add=True)

        def wait_scatter(j, b):
            pltpu.make_async_copy(vrows.at[b], acc.at[vdst.at[j]],
                                  ssem[b]).wait()
            if with_deg:
                pltpu.make_async_copy(vones, accd.at[vdst.at[j]],
                                      osem[b]).wait()

        for b in range(NBUF):
            fire_gather(b, b)

        def group(gi, _):
            j0 = gi * NBUF
            for b in range(NBUF):
                wait_gather(j0 + b, b)
                fire_scatter(j0 + b, b)
            for b in range(NBUF):
                wait_scatter(j0 + b, b)
                fire_gather(j0 + NBUF + b, b)
            return 0
        lax.fori_loop(0, NGROUP - 1, group, 0)

        j0 = (NGROUP - 1) * NBUF
        for b in range(NBUF):
            wait_gather(j0 + b, b)
            fire_scatter(j0 + b, b)
        for b in range(NBUF):
            wait_scatter(j0 + b, b)
        plsc.subcore_barrier()

        def writeout(nrows):
            base = s * RPT
            pltpu.sync_copy(acc.at[pl.ds(base, nrows)],
                            vstage.at[pl.ds(0, nrows)])
            if with_deg:
                pltpu.sync_copy(
                    vstage.at[pl.ds(0, nrows)],
                    out_hbm.at[c, pl.ds(base, nrows), pl.ds(0, width)])
                pltpu.sync_copy(accd.at[pl.ds(base, nrows)],
                                vstaged.at[pl.ds(0, nrows)])
                pltpu.sync_copy(
                    vstaged.at[pl.ds(0, nrows)],
                    out_hbm.at[c, pl.ds(base, nrows), pl.ds(width, 16)])
            else:
                pltpu.sync_copy(
                    vstage.at[pl.ds(0, nrows)],
                    out_hbm.at[c, pl.ds(base, nrows), pl.ds(0, width)])

        last = N - (NS - 1) * RPT

        @pl.when(s < NS - 1)
        def _():
            writeout(RPT)

        @pl.when(s == NS - 1)
        def _():
            writeout(last)

    return pl.kernel(body, out_type=out_type, mesh=mesh,
                     scratch_types=scratch,
                     compiler_params=pltpu.CompilerParams(
                         use_tc_tiling_on_sc=False))


def _seg_sum_h(table, src, dst):
    (segdeg,) = _make_seg_sum(ED, True)(table, src, dst)
    return segdeg


def _seg_sum_s(table, src, dst):
    (mseg,) = _make_seg_sum(KP, False)(table, src, dst)
    return mseg


_GRID = 5
_ROWS = N // _GRID


def _embed_body(x_ref, w_ref, b_ref, h_ref):
    h_ref[:, :ED] = _dot(x_ref[...], w_ref[...]) + b_ref[...]
    h_ref[:, ED:] = jnp.zeros((h_ref.shape[0], OW - ED), jnp.float32)


@jax.jit
def _tc_embed(x, w, b):
    return pl.pallas_call(
        _embed_body,
        grid=(_GRID,),
        in_specs=[
            pl.BlockSpec((_ROWS, DF), lambda i: (i, 0)),
            pl.BlockSpec((DF, ED), lambda i: (0, 0)),
            pl.BlockSpec((1, ED), lambda i: (0, 0)),
        ],
        out_specs=pl.BlockSpec((_ROWS, OW), lambda i: (i, 0)),
        out_shape=jax.ShapeDtypeStruct((N, OW), jnp.float32),
    )(x, w, b)


def _mid_body(h_ref, sd_ref, w1s_ref, w1n_ref, ws1s_ref, ws1n_ref,
              s1_ref, x1_ref, x1_acc):
    i = pl.program_id(0)
    h = h_ref[:, :ED]
    seg = sd_ref[0, :, :ED] + sd_ref[1, :, :ED]
    deg = sd_ref[0, :, ED:ED + 1] + sd_ref[1, :, ED:ED + 1]
    agg = seg / jnp.maximum(deg, 1.0)
    z1 = jnp.maximum(_dot(h, w1s_ref[...]) + _dot(agg, w1n_ref[...]), 0.0)
    logits = _dot(h, ws1s_ref[...]) + _dot(agg, ws1n_ref[...])
    col = lax.broadcasted_iota(jnp.int32, logits.shape, 1)
    logits = jnp.where(col >= K, -1e30, logits)
    m = jnp.max(logits, axis=1, keepdims=True)
    e = jnp.exp(logits - m)
    s1 = e / jnp.sum(e, axis=1, keepdims=True)
    s1_ref[:, :KP] = s1
    s1_ref[:, KP:] = jnp.zeros((s1.shape[0], OW - KP), jnp.float32)

    @pl.when(i == 0)
    def _():
        x1_acc[...] = jnp.zeros_like(x1_acc)

    x1_acc[...] += _dot_t(s1, z1)

    @pl.when(i == _GRID - 1)
    def _():
        x1_ref[...] = x1_acc[...]


@jax.jit
def _tc_mid(h, segdeg, w1s, w1n, ws1s, ws1n):
    return pl.pallas_call(
        _mid_body,
        grid=(_GRID,),
        in_specs=[
            pl.BlockSpec((_ROWS, OW), lambda i: (i, 0)),
            pl.BlockSpec((NC, _ROWS, OW), lambda i: (0, i, 0)),
            pl.BlockSpec((ED, ED), lambda i: (0, 0)),
            pl.BlockSpec((ED, ED), lambda i: (0, 0)),
            pl.BlockSpec((ED, KP), lambda i: (0, 0)),
            pl.BlockSpec((ED, KP), lambda i: (0, 0)),
        ],
        out_specs=[
            pl.BlockSpec((_ROWS, OW), lambda i: (i, 0)),
            pl.BlockSpec((KP, ED), lambda i: (0, 0)),
        ],
        out_shape=[
            jax.ShapeDtypeStruct((N, OW), jnp.float32),
            jax.ShapeDtypeStruct((KP, ED), jnp.float32),
        ],
        scratch_shapes=[pltpu.VMEM((KP, ED), jnp.float32)],
    )(h, segdeg, w1s, w1n, ws1s, ws1n)


def _tail_body(m_ref, s1_ref, x1_ref, w2s_ref, w2n_ref, ws2s_ref, ws2n_ref,
               wh1s_ref, wh1n_ref, wh2s_ref, wh2n_ref,
               zout_ref, s2_ref, a1_acc):
    i = pl.program_id(0)

    @pl.when(i == 0)
    def _():
        a1_acc[...] = jnp.zeros_like(a1_acc)

    m = m_ref[0, :, :KP] + m_ref[1, :, :KP]
    a1_acc[...] += _dot_t(m, s1_ref[:, :KP])

    @pl.when(i == _GRID - 1)
    def _():
        a1 = a1_acc[...]
        x1 = x1_ref[...]
        d1 = jnp.maximum(jnp.sum(a1, axis=1, keepdims=True), 1.0)
        agg1 = _dot(a1, x1) / d1
        z2 = jnp.maximum(_dot(x1, w2s_ref[...]) + _dot(agg1, w2n_ref[...]),
                         0.0)
        l2 = _dot(x1, ws2s_ref[...]) + _dot(agg1, ws2n_ref[...])
        col = lax.broadcasted_iota(jnp.int32, l2.shape, 1)
        l2 = jnp.where(col >= K, -1e30, l2)
        mx = jnp.max(l2, axis=1, keepdims=True)
        ex = jnp.exp(l2 - mx)
        s2 = ex / jnp.sum(ex, axis=1, keepdims=True)
        s2_ref[...] = s2
        x2 = _dot_t(s2, z2)
        a2 = _dot(_dot_t(s2, a1), s2)
        d2 = jnp.maximum(jnp.sum(a2, axis=1, keepdims=True), 1.0)
        h1 = jnp.maximum(
            _dot(x2, wh1s_ref[...]) + _dot(_dot(a2, x2) / d2, wh1n_ref[...]),
            0.0)
        zout_ref[...] = jnp.maximum(
            _dot(h1, wh2s_ref[...]) + _dot(_dot(a2, h1) / d2, wh2n_ref[...]),
            0.0)


@jax.jit
def _tc_tail(mseg, s1, x1, w2s, w2n, ws2s, ws2n, wh1s, wh1n, wh2s, wh2n):
    small = lambda r, c: pl.BlockSpec((r, c), lambda i: (0, 0))
    return pl.pallas_call(
        _tail_body,
        grid=(_GRID,),
        in_specs=[
            pl.BlockSpec((NC, _ROWS, OW), lambda i: (0, i, 0)),
            pl.BlockSpec((_ROWS, OW), lambda i: (i, 0)),
            small(KP, ED),
            small(ED, ED), small(ED, ED), small(ED, KP), small(ED, KP),
            small(ED, HD), small(ED, HD), small(HD, HD), small(HD, HD),
        ],
        out_specs=[
            pl.BlockSpec((KP, HD), lambda i: (0, 0)),
            pl.BlockSpec((KP, KP), lambda i: (0, 0)),
        ],
        out_shape=[
            jax.ShapeDtypeStruct((KP, HD), jnp.float32),
            jax.ShapeDtypeStruct((KP, KP), jnp.float32),
        ],
        scratch_shapes=[pltpu.VMEM((KP, KP), jnp.float32)],
    )(mseg, s1, x1, w2s, w2n, ws2s, ws2n, wh1s, wh1n, wh2s, wh2n)


def kernel(x_note, edge_index, batch, W_embed, b_embed, W1_self, W1_neigh,
           Ws1_self, Ws1_neigh, W2_self, W2_neigh, Ws2_self, Ws2_neigh,
           Wh1_self, Wh1_neigh, Wh2_self, Wh2_neigh):
    del batch
    npad = EPP - EPW
    src = jnp.pad(edge_index[0].reshape(NW, EPW), ((0, 0), (0, npad)),
                  constant_values=0).reshape(NW, NCHUNK, CHUNK)
    pad_dst = jnp.broadcast_to(N + jnp.arange(npad, dtype=jnp.int32),
                               (NW, npad))
    dst = jnp.concatenate([edge_index[1].reshape(NW, EPW), pad_dst],
                          axis=1).reshape(NW, NCHUNK, CHUNK)
    pad1 = lambda w: jnp.pad(w, ((0, 0), (0, KP - K)))

    h = _tc_embed(x_note, W_embed, b_embed.reshape(1, ED))
    segdeg = _seg_sum_h(h.reshape(N * (OW // ED), ED), src, dst)
    s1p, x1 = _tc_mid(h, segdeg, W1_self, W1_neigh,
                      pad1(Ws1_self), pad1(Ws1_neigh))
    mseg = _seg_sum_s(s1p.reshape(N * (OW // KP), KP), src, dst)
    zout, s2p = _tc_tail(mseg, s1p, x1, W2_self, W2_neigh, pad1(Ws2_self),
                         pad1(Ws2_neigh), Wh1_self, Wh1_neigh, Wh2_self,
                         Wh2_neigh)
    return (zout[:K], s1p[:, :K], s2p[:K, :K])


# speedup vs baseline: 1.0044x; 1.0044x over previous
import functools

import jax
import jax.numpy as jnp
from jax import lax
from jax.experimental import pallas as pl
from jax.experimental.pallas import tpu as pltpu
from jax.experimental.pallas import tpu_sc as plsc

N = 10000
E = 320000
DF = 128
ED = 32
HD = 200
K = 15
KP = 16

NC = 2
NS = 16
NW = NC * NS
EPW = E // NW
CHUNK = 96
EPP = 10080
NCHUNK = EPP // CHUNK
NBUF = 5
NGROUP = NCHUNK // NBUF
OW = 128
NP = 10240
RPT = NP // NS

def _dot(a, b):
    return jnp.dot(a, b, preferred_element_type=jnp.float32)


def _dot_t(a, b):
    return lax.dot_general(a, b, (((0,), (0,)), ((), ())),
                           preferred_element_type=jnp.float32)


@functools.lru_cache(maxsize=None)
def _make_seg_sum(width, with_deg):
    mesh = plsc.VectorSubcoreMesh(core_axis_name="c", subcore_axis_name="s",
                                  num_cores=NC, num_subcores=NS)
    out_type = [jax.ShapeDtypeStruct((NC, N, OW), jnp.float32)]
    nsem = 3 * NBUF if with_deg else 2 * NBUF
    scratch = [
        pltpu.VMEM((NCHUNK, CHUNK), jnp.int32),
        pltpu.VMEM((NCHUNK, CHUNK), jnp.int32),
        pltpu.VMEM((NBUF, CHUNK, width), jnp.float32),
        pltpu.VMEM((RPT, width), jnp.float32),
        pltpu.VMEM_SHARED((NP, width), jnp.float32),
    ] + [pltpu.SemaphoreType.DMA] * nsem
    if with_deg:
        scratch += [
            pltpu.VMEM((CHUNK, 16), jnp.float32),
            pltpu.VMEM((RPT, 16), jnp.float32),
            pltpu.VMEM_SHARED((NP, 16), jnp.float32),
        ]

    def body(table_hbm, src_hbm, dst_hbm, *refs):
        if with_deg:
            (out_hbm, vsrc, vdst, vrows, vstage, acc) = refs[:6]
            sems = refs[6:6 + nsem]
            vones, vstaged, accd = refs[6 + nsem:]
            gsem, ssem, osem = (sems[:NBUF], sems[NBUF:2 * NBUF],
                                sems[2 * NBUF:])
        else:
            (out_hbm, vsrc, vdst, vrows, vstage, acc) = refs[:6]
            sems = refs[6:6 + nsem]
            gsem, ssem = sems[:NBUF], sems[NBUF:]
        c = lax.axis_index("c")
        s = lax.axis_index("s")
        wid = c * NS + s
        zeros16 = jnp.zeros((16,), jnp.float32)

        def zrow(i, _):
            for j0 in range(width // 16):
                vstage[i, pl.ds(j0 * 16, 16)] = zeros16
            if with_deg:
                vstaged[i, pl.ds(0, 16)] = zeros16
            return 0
        lax.fori_loop(0, RPT, zrow, 0)
        pltpu.sync_copy(vstage, acc.at[pl.ds(s * RPT, RPT)])
        if with_deg:
            pltpu.sync_copy(vstaged, accd.at[pl.ds(s * RPT, RPT)])
            ones16 = jnp.ones((16,), jnp.float32)

            def orow(i, _):
                vones[i, pl.ds(0, 16)] = ones16
                return 0
            lax.fori_loop(0, CHUNK, orow, 0)
        plsc.subcore_barrier()

        pltpu.sync_copy(src_hbm.at[wid], vsrc)
        pltpu.sync_copy(dst_hbm.at[wid], vdst)

        factor = OW // width

        def srow(i, _):
            for k in range(CHUNK // 16):
                vsrc[i, pl.ds(k * 16, 16)] = vsrc[i, pl.ds(k * 16, 16)] * factor
            return 0
        lax.fori_loop(0, NCHUNK, srow, 0)

        def fire_gather(j, b):
            pltpu.async_copy(table_hbm.at[vsrc.at[j]], vrows.at[b], gsem[b])

        def wait_gather(j, b):
            pltpu.make_async_copy(table_hbm.at[vsrc.at[j]], vrows.at[b],
                                  gsem[b]).wait()

        def fire_scatter(j, b):
            pltpu.async_copy(vrows.at[b], acc.at[vdst.at[j]], ssem[b],
                             add=True)
            if with_deg:
                pltpu.async_copy(vones, accd.at[vdst.at[j]], osem[b],
                                 add=True)

        def wait_scatter(j, b):
            pltpu.make_async_copy(vrows.at[b], acc.at[vdst.at[j]],
                                  ssem[b]).wait()
            if with_deg:
                pltpu.make_async_copy(vones, accd.at[vdst.at[j]],
                                      osem[b]).wait()

        for b in range(NBUF):
            fire_gather(b, b)

        def group(gi, _):
            j0 = gi * NBUF
            for b in range(NBUF):
                wait_gather(j0 + b, b)
                fire_scatter(j0 + b, b)
            for b in range(NBUF):
                wait_scatter(j0 + b, b)
                fire_gather(j0 + NBUF + b, b)
            return 0
        lax.fori_loop(0, NGROUP - 1, group, 0)

        j0 = (NGROUP - 1) * NBUF
        for b in range(NBUF):
            wait_gather(j0 + b, b)
            fire_scatter(j0 + b, b)
        for b in range(NBUF):
            wait_scatter(j0 + b, b)
        plsc.subcore_barrier()

        def writeout(nrows):
            base = s * RPT
            pltpu.sync_copy(acc.at[pl.ds(base, nrows)],
                            vstage.at[pl.ds(0, nrows)])
            if with_deg:
                pltpu.sync_copy(
                    vstage.at[pl.ds(0, nrows)],
                    out_hbm.at[c, pl.ds(base, nrows), pl.ds(0, width)])
                pltpu.sync_copy(accd.at[pl.ds(base, nrows)],
                                vstaged.at[pl.ds(0, nrows)])
                pltpu.sync_copy(
                    vstaged.at[pl.ds(0, nrows)],
                    out_hbm.at[c, pl.ds(base, nrows), pl.ds(width, 16)])
            else:
                pltpu.sync_copy(
                    vstage.at[pl.ds(0, nrows)],
                    out_hbm.at[c, pl.ds(base, nrows), pl.ds(0, width)])

        last = N - (NS - 1) * RPT

        @pl.when(s < NS - 1)
        def _():
            writeout(RPT)

        @pl.when(s == NS - 1)
        def _():
            writeout(last)

    return pl.kernel(body, out_type=out_type, mesh=mesh,
                     scratch_types=scratch,
                     compiler_params=pltpu.CompilerParams(
                         use_tc_tiling_on_sc=False))


def _seg_sum_h(table, src, dst):
    (segdeg,) = _make_seg_sum(ED, True)(table, src, dst)
    return segdeg


def _seg_sum_s(table, src, dst):
    (mseg,) = _make_seg_sum(KP, False)(table, src, dst)
    return mseg


_GRID = 5
_ROWS = N // _GRID


def _embed_body(x_ref, w_ref, b_ref, h_ref):
    h_ref[:, :ED] = _dot(x_ref[...], w_ref[...]) + b_ref[...]
    h_ref[:, ED:] = jnp.zeros((h_ref.shape[0], OW - ED), jnp.float32)


@jax.jit
def _tc_embed(x, w, b):
    return pl.pallas_call(
        _embed_body,
        grid=(_GRID,),
        in_specs=[
            pl.BlockSpec((_ROWS, DF), lambda i: (i, 0)),
            pl.BlockSpec((DF, ED), lambda i: (0, 0)),
            pl.BlockSpec((1, ED), lambda i: (0, 0)),
        ],
        out_specs=pl.BlockSpec((_ROWS, OW), lambda i: (i, 0)),
        out_shape=jax.ShapeDtypeStruct((N, OW), jnp.float32),
    )(x, w, b)


def _mid_body(h_ref, sd_ref, w1s_ref, w1n_ref, ws1s_ref, ws1n_ref,
              s1_ref, x1_ref, x1_acc):
    i = pl.program_id(0)
    h = h_ref[:, :ED]
    seg = sd_ref[0, :, :ED] + sd_ref[1, :, :ED]
    deg = sd_ref[0, :, ED:ED + 1] + sd_ref[1, :, ED:ED + 1]
    agg = seg / jnp.maximum(deg, 1.0)
    z1 = jnp.maximum(_dot(h, w1s_ref[...]) + _dot(agg, w1n_ref[...]), 0.0)
    logits = _dot(h, ws1s_ref[...]) + _dot(agg, ws1n_ref[...])
    col = lax.broadcasted_iota(jnp.int32, logits.shape, 1)
    logits = jnp.where(col >= K, -1e30, logits)
    m = jnp.max(logits, axis=1, keepdims=True)
    e = jnp.exp(logits - m)
    s1 = e / jnp.sum(e, axis=1, keepdims=True)
    s1_ref[:, :KP] = s1
    s1_ref[:, KP:] = jnp.zeros((s1.shape[0], OW - KP), jnp.float32)

    @pl.when(i == 0)
    def _():
        x1_acc[...] = jnp.zeros_like(x1_acc)

    x1_acc[...] += _dot_t(s1, z1)

    @pl.when(i == _GRID - 1)
    def _():
        x1_ref[...] = x1_acc[...]


@jax.jit
def _tc_mid(h, segdeg, w1s, w1n, ws1s, ws1n):
    return pl.pallas_call(
        _mid_body,
        grid=(_GRID,),
        in_specs=[
            pl.BlockSpec((_ROWS, OW), lambda i: (i, 0)),
            pl.BlockSpec((NC, _ROWS, OW), lambda i: (0, i, 0)),
            pl.BlockSpec((ED, ED), lambda i: (0, 0)),
            pl.BlockSpec((ED, ED), lambda i: (0, 0)),
            pl.BlockSpec((ED, KP), lambda i: (0, 0)),
            pl.BlockSpec((ED, KP), lambda i: (0, 0)),
        ],
        out_specs=[
            pl.BlockSpec((_ROWS, OW), lambda i: (i, 0)),
            pl.BlockSpec((KP, ED), lambda i: (0, 0)),
        ],
        out_shape=[
            jax.ShapeDtypeStruct((N, OW), jnp.float32),
            jax.ShapeDtypeStruct((KP, ED), jnp.float32),
        ],
        scratch_shapes=[pltpu.VMEM((KP, ED), jnp.float32)],
    )(h, segdeg, w1s, w1n, ws1s, ws1n)


def _tail_body(m_ref, s1_ref, x1_ref, w2s_ref, w2n_ref, ws2s_ref, ws2n_ref,
               wh1s_ref, wh1n_ref, wh2s_ref, wh2n_ref,
               zout_ref, s2_ref, a1_acc):
    i = pl.program_id(0)

    @pl.when(i == 0)
    def _():
        a1_acc[...] = jnp.zeros_like(a1_acc)

    m = m_ref[0, :, :KP] + m_ref[1, :, :KP]
    a1_acc[...] += _dot_t(m, s1_ref[:, :KP])

    @pl.when(i == _GRID - 1)
    def _():
        a1 = a1_acc[...]
        x1 = x1_ref[...]
        d1 = jnp.maximum(jnp.sum(a1, axis=1, keepdims=True), 1.0)
        agg1 = _dot(a1, x1) / d1
        z2 = jnp.maximum(_dot(x1, w2s_ref[...]) + _dot(agg1, w2n_ref[...]),
                         0.0)
        l2 = _dot(x1, ws2s_ref[...]) + _dot(agg1, ws2n_ref[...])
        col = lax.broadcasted_iota(jnp.int32, l2.shape, 1)
        l2 = jnp.where(col >= K, -1e30, l2)
        mx = jnp.max(l2, axis=1, keepdims=True)
        ex = jnp.exp(l2 - mx)
        s2 = ex / jnp.sum(ex, axis=1, keepdims=True)
        s2_ref[...] = s2
        x2 = _dot_t(s2, z2)
        a2 = _dot(_dot_t(s2, a1), s2)
        d2 = jnp.maximum(jnp.sum(a2, axis=1, keepdims=True), 1.0)
        h1 = jnp.maximum(
            _dot(x2, wh1s_ref[...]) + _dot(_dot(a2, x2) / d2, wh1n_ref[...]),
            0.0)
        zout_ref[...] = jnp.maximum(
            _dot(h1, wh2s_ref[...]) + _dot(_dot(a2, h1) / d2, wh2n_ref[...]),
            0.0)


@jax.jit
def _tc_tail(mseg, s1, x1, w2s, w2n, ws2s, ws2n, wh1s, wh1n, wh2s, wh2n):
    small = lambda r, c: pl.BlockSpec((r, c), lambda i: (0, 0))
    return pl.pallas_call(
        _tail_body,
        grid=(_GRID,),
        in_specs=[
            pl.BlockSpec((NC, _ROWS, OW), lambda i: (0, i, 0)),
            pl.BlockSpec((_ROWS, OW), lambda i: (i, 0)),
            small(KP, ED),
            small(ED, ED), small(ED, ED), small(ED, KP), small(ED, KP),
            small(ED, HD), small(ED, HD), small(HD, HD), small(HD, HD),
        ],
        out_specs=[
            pl.BlockSpec((KP, HD), lambda i: (0, 0)),
            pl.BlockSpec((KP, KP), lambda i: (0, 0)),
        ],
        out_shape=[
            jax.ShapeDtypeStruct((KP, HD), jnp.float32),
            jax.ShapeDtypeStruct((KP, KP), jnp.float32),
        ],
        scratch_shapes=[pltpu.VMEM((KP, KP), jnp.float32)],
    )(mseg, s1, x1, w2s, w2n, ws2s, ws2n, wh1s, wh1n, wh2s, wh2n)


def kernel(x_note, edge_index, batch, W_embed, b_embed, W1_self, W1_neigh,
           Ws1_self, Ws1_neigh, W2_self, W2_neigh, Ws2_self, Ws2_neigh,
           Wh1_self, Wh1_neigh, Wh2_self, Wh2_neigh):
    del batch
    npad = EPP - EPW
    src = jnp.pad(edge_index[0].reshape(NW, EPW), ((0, 0), (0, npad)),
                  constant_values=0).reshape(NW, NCHUNK, CHUNK)
    pad_dst = jnp.broadcast_to(N + jnp.arange(npad, dtype=jnp.int32),
                               (NW, npad))
    dst = jnp.concatenate([edge_index[1].reshape(NW, EPW), pad_dst],
                          axis=1).reshape(NW, NCHUNK, CHUNK)
    pad1 = lambda w: jnp.pad(w, ((0, 0), (0, KP - K)))

    h = _tc_embed(x_note, W_embed, b_embed.reshape(1, ED))
    segdeg = _seg_sum_h(h.reshape(N * (OW // ED), ED), src, dst)
    s1p, x1 = _tc_mid(h, segdeg, W1_self, W1_neigh,
                      pad1(Ws1_self), pad1(Ws1_neigh))
    mseg = _seg_sum_s(s1p.reshape(N * (OW // KP), KP), src, dst)
    zout, s2p = _tc_tail(mseg, s1p, x1, W2_self, W2_neigh, pad1(Ws2_self),
                         pad1(Ws2_neigh), Wh1_self, Wh1_neigh, Wh2_self,
                         Wh2_neigh)
    return (zout[:K], s1p[:, :K], s2p[:K, :K])


# speedup vs baseline: 1.1129x; 1.1080x over previous
import functools

import jax
import jax.numpy as jnp
from jax import lax
from jax.experimental import pallas as pl
from jax.experimental.pallas import tpu as pltpu
from jax.experimental.pallas import tpu_sc as plsc

N = 10000
E = 320000
DF = 128
ED = 32
HD = 200
K = 15
KP = 16

NC = 2
NS = 16
NW = NC * NS
EPW = E // NW
CHUNK = 80
NCHUNK = EPW // CHUNK
NBUF = 5
NGROUP = NCHUNK // NBUF
OW = 128
NP = 10240
RPT = NP // NS

def _dot(a, b):
    return jnp.dot(a, b, preferred_element_type=jnp.float32)


def _dot_t(a, b):
    return lax.dot_general(a, b, (((0,), (0,)), ((), ())),
                           preferred_element_type=jnp.float32)


@functools.lru_cache(maxsize=None)
def _make_seg_sum(width, with_deg):
    mesh = plsc.VectorSubcoreMesh(core_axis_name="c", subcore_axis_name="s",
                                  num_cores=NC, num_subcores=NS)
    out_type = [jax.ShapeDtypeStruct((NC, N, OW), jnp.float32)]
    nsem = 3 * NBUF if with_deg else 2 * NBUF
    scratch = [
        pltpu.VMEM((NCHUNK, CHUNK), jnp.int32),
        pltpu.VMEM((NCHUNK, CHUNK), jnp.int32),
        pltpu.VMEM((NBUF, CHUNK, width), jnp.float32),
        pltpu.VMEM((RPT, width), jnp.float32),
        pltpu.VMEM_SHARED((NP, width), jnp.float32),
    ] + [pltpu.SemaphoreType.DMA] * nsem
    if with_deg:
        scratch += [
            pltpu.VMEM((CHUNK, 16), jnp.float32),
            pltpu.VMEM((RPT, 16), jnp.float32),
            pltpu.VMEM_SHARED((NP, 16), jnp.float32),
        ]

    def body(table_hbm, src_hbm, dst_hbm, *refs):
        if with_deg:
            (out_hbm, vsrc, vdst, vrows, vstage, acc) = refs[:6]
            sems = refs[6:6 + nsem]
            vones, vstaged, accd = refs[6 + nsem:]
            gsem, ssem, osem = (sems[:NBUF], sems[NBUF:2 * NBUF],
                                sems[2 * NBUF:])
        else:
            (out_hbm, vsrc, vdst, vrows, vstage, acc) = refs[:6]
            sems = refs[6:6 + nsem]
            gsem, ssem = sems[:NBUF], sems[NBUF:]
        c = lax.axis_index("c")
        s = lax.axis_index("s")
        wid = c * NS + s
        zeros16 = jnp.zeros((16,), jnp.float32)

        def zrow(i, _):
            for j0 in range(width // 16):
                vstage[i, pl.ds(j0 * 16, 16)] = zeros16
            if with_deg:
                vstaged[i, pl.ds(0, 16)] = zeros16
            return 0
        lax.fori_loop(0, RPT, zrow, 0)
        pltpu.sync_copy(vstage, acc.at[pl.ds(s * RPT, RPT)])
        if with_deg:
            pltpu.sync_copy(vstaged, accd.at[pl.ds(s * RPT, RPT)])
            ones16 = jnp.ones((16,), jnp.float32)

            def orow(i, _):
                vones[i, pl.ds(0, 16)] = ones16
                return 0
            lax.fori_loop(0, CHUNK, orow, 0)
        plsc.subcore_barrier()

        pltpu.sync_copy(src_hbm.at[wid], vsrc)
        pltpu.sync_copy(dst_hbm.at[wid], vdst)

        factor = OW // width

        def srow(i, _):
            for k in range(CHUNK // 16):
                vsrc[i, pl.ds(k * 16, 16)] = vsrc[i, pl.ds(k * 16, 16)] * factor
            return 0
        lax.fori_loop(0, NCHUNK, srow, 0)

        def fire_gather(j, b):
            pltpu.async_copy(table_hbm.at[vsrc.at[j]], vrows.at[b], gsem[b])

        def wait_gather(j, b):
            pltpu.make_async_copy(table_hbm.at[vsrc.at[j]], vrows.at[b],
                                  gsem[b]).wait()

        def fire_scatter(j, b):
            pltpu.async_copy(vrows.at[b], acc.at[vdst.at[j]], ssem[b],
                             add=True)
            if with_deg:
                pltpu.async_copy(vones, accd.at[vdst.at[j]], osem[b],
                                 add=True)

        def wait_scatter(j, b):
            pltpu.make_async_copy(vrows.at[b], acc.at[vdst.at[j]],
                                  ssem[b]).wait()
            if with_deg:
                pltpu.make_async_copy(vones, accd.at[vdst.at[j]],
                                      osem[b]).wait()

        for b in range(NBUF):
            fire_gather(b, b)

        def group(gi, _):
            j0 = gi * NBUF
            for b in range(NBUF):
                wait_gather(j0 + b, b)
                fire_scatter(j0 + b, b)
            for b in range(NBUF):
                wait_scatter(j0 + b, b)
                fire_gather(j0 + NBUF + b, b)
            return 0
        lax.fori_loop(0, NGROUP - 1, group, 0)

        j0 = (NGROUP - 1) * NBUF
        for b in range(NBUF):
            wait_gather(j0 + b, b)
            fire_scatter(j0 + b, b)
        for b in range(NBUF):
            wait_scatter(j0 + b, b)
        plsc.subcore_barrier()

        def writeout(nrows):
            base = s * RPT
            pltpu.sync_copy(acc.at[pl.ds(base, nrows)],
                            vstage.at[pl.ds(0, nrows)])
            if with_deg:
                pltpu.sync_copy(
                    vstage.at[pl.ds(0, nrows)],
                    out_hbm.at[c, pl.ds(base, nrows), pl.ds(0, width)])
                pltpu.sync_copy(accd.at[pl.ds(base, nrows)],
                                vstaged.at[pl.ds(0, nrows)])
                pltpu.sync_copy(
                    vstaged.at[pl.ds(0, nrows)],
                    out_hbm.at[c, pl.ds(base, nrows), pl.ds(width, 16)])
            else:
                pltpu.sync_copy(
                    vstage.at[pl.ds(0, nrows)],
                    out_hbm.at[c, pl.ds(base, nrows), pl.ds(0, width)])

        last = N - (NS - 1) * RPT

        @pl.when(s < NS - 1)
        def _():
            writeout(RPT)

        @pl.when(s == NS - 1)
        def _():
            writeout(last)

    return pl.kernel(body, out_type=out_type, mesh=mesh,
                     scratch_types=scratch,
                     compiler_params=pltpu.CompilerParams(
                         use_tc_tiling_on_sc=False))


def _seg_sum_h(table, src, dst):
    (segdeg,) = _make_seg_sum(2 * ED, False)(table, src, dst)
    return segdeg


def _seg_sum_s(table, src, dst):
    (mseg,) = _make_seg_sum(KP, False)(table, src, dst)
    return mseg


_GRID = 5
_ROWS = N // _GRID


def _embed_body(x_ref, w_ref, b_ref, h_ref):
    h_ref[:, :ED] = _dot(x_ref[...], w_ref[...]) + b_ref[...]
    h_ref[:, ED:ED + 16] = jnp.ones((h_ref.shape[0], 16), jnp.float32)
    h_ref[:, ED + 16:] = jnp.zeros((h_ref.shape[0], OW - ED - 16),
                                   jnp.float32)


@jax.jit
def _tc_embed(x, w, b):
    return pl.pallas_call(
        _embed_body,
        grid=(_GRID,),
        in_specs=[
            pl.BlockSpec((_ROWS, DF), lambda i: (i, 0)),
            pl.BlockSpec((DF, ED), lambda i: (0, 0)),
            pl.BlockSpec((1, ED), lambda i: (0, 0)),
        ],
        out_specs=pl.BlockSpec((_ROWS, OW), lambda i: (i, 0)),
        out_shape=jax.ShapeDtypeStruct((N, OW), jnp.float32),
    )(x, w, b)


def _mid_body(h_ref, sd_ref, w1s_ref, w1n_ref, ws1s_ref, ws1n_ref,
              s1_ref, x1_ref, x1_acc):
    i = pl.program_id(0)
    h = h_ref[:, :ED]
    seg = sd_ref[0, :, :ED] + sd_ref[1, :, :ED]
    deg = sd_ref[0, :, ED:ED + 1] + sd_ref[1, :, ED:ED + 1]
    agg = seg / jnp.maximum(deg, 1.0)
    z1 = jnp.maximum(_dot(h, w1s_ref[...]) + _dot(agg, w1n_ref[...]), 0.0)
    logits = _dot(h, ws1s_ref[...]) + _dot(agg, ws1n_ref[...])
    col = lax.broadcasted_iota(jnp.int32, logits.shape, 1)
    logits = jnp.where(col >= K, -1e30, logits)
    m = jnp.max(logits, axis=1, keepdims=True)
    e = jnp.exp(logits - m)
    s1 = e / jnp.sum(e, axis=1, keepdims=True)
    s1_ref[:, :KP] = s1
    s1_ref[:, KP:] = jnp.zeros((s1.shape[0], OW - KP), jnp.float32)

    @pl.when(i == 0)
    def _():
        x1_acc[...] = jnp.zeros_like(x1_acc)

    x1_acc[...] += _dot_t(s1, z1)

    @pl.when(i == _GRID - 1)
    def _():
        x1_ref[...] = x1_acc[...]


@jax.jit
def _tc_mid(h, segdeg, w1s, w1n, ws1s, ws1n):
    return pl.pallas_call(
        _mid_body,
        grid=(_GRID,),
        in_specs=[
            pl.BlockSpec((_ROWS, OW), lambda i: (i, 0)),
            pl.BlockSpec((NC, _ROWS, OW), lambda i: (0, i, 0)),
            pl.BlockSpec((ED, ED), lambda i: (0, 0)),
            pl.BlockSpec((ED, ED), lambda i: (0, 0)),
            pl.BlockSpec((ED, KP), lambda i: (0, 0)),
            pl.BlockSpec((ED, KP), lambda i: (0, 0)),
        ],
        out_specs=[
            pl.BlockSpec((_ROWS, OW), lambda i: (i, 0)),
            pl.BlockSpec((KP, ED), lambda i: (0, 0)),
        ],
        out_shape=[
            jax.ShapeDtypeStruct((N, OW), jnp.float32),
            jax.ShapeDtypeStruct((KP, ED), jnp.float32),
        ],
        scratch_shapes=[pltpu.VMEM((KP, ED), jnp.float32)],
    )(h, segdeg, w1s, w1n, ws1s, ws1n)


def _tail_body(m_ref, s1_ref, x1_ref, w2s_ref, w2n_ref, ws2s_ref, ws2n_ref,
               wh1s_ref, wh1n_ref, wh2s_ref, wh2n_ref,
               zout_ref, s2_ref, a1_acc):
    i = pl.program_id(0)

    @pl.when(i == 0)
    def _():
        a1_acc[...] = jnp.zeros_like(a1_acc)

    m = m_ref[0, :, :KP] + m_ref[1, :, :KP]
    a1_acc[...] += _dot_t(m, s1_ref[:, :KP])

    @pl.when(i == _GRID - 1)
    def _():
        a1 = a1_acc[...]
        x1 = x1_ref[...]
        d1 = jnp.maximum(jnp.sum(a1, axis=1, keepdims=True), 1.0)
        agg1 = _dot(a1, x1) / d1
        z2 = jnp.maximum(_dot(x1, w2s_ref[...]) + _dot(agg1, w2n_ref[...]),
                         0.0)
        l2 = _dot(x1, ws2s_ref[...]) + _dot(agg1, ws2n_ref[...])
        col = lax.broadcasted_iota(jnp.int32, l2.shape, 1)
        l2 = jnp.where(col >= K, -1e30, l2)
        mx = jnp.max(l2, axis=1, keepdims=True)
        ex = jnp.exp(l2 - mx)
        s2 = ex / jnp.sum(ex, axis=1, keepdims=True)
        s2_ref[...] = s2
        x2 = _dot_t(s2, z2)
        a2 = _dot(_dot_t(s2, a1), s2)
        d2 = jnp.maximum(jnp.sum(a2, axis=1, keepdims=True), 1.0)
        h1 = jnp.maximum(
            _dot(x2, wh1s_ref[...]) + _dot(_dot(a2, x2) / d2, wh1n_ref[...]),
            0.0)
        zout_ref[...] = jnp.maximum(
            _dot(h1, wh2s_ref[...]) + _dot(_dot(a2, h1) / d2, wh2n_ref[...]),
            0.0)


@jax.jit
def _tc_tail(mseg, s1, x1, w2s, w2n, ws2s, ws2n, wh1s, wh1n, wh2s, wh2n):
    small = lambda r, c: pl.BlockSpec((r, c), lambda i: (0, 0))
    return pl.pallas_call(
        _tail_body,
        grid=(_GRID,),
        in_specs=[
            pl.BlockSpec((NC, _ROWS, OW), lambda i: (0, i, 0)),
            pl.BlockSpec((_ROWS, OW), lambda i: (i, 0)),
            small(KP, ED),
            small(ED, ED), small(ED, ED), small(ED, KP), small(ED, KP),
            small(ED, HD), small(ED, HD), small(HD, HD), small(HD, HD),
        ],
        out_specs=[
            pl.BlockSpec((KP, HD), lambda i: (0, 0)),
            pl.BlockSpec((KP, KP), lambda i: (0, 0)),
        ],
        out_shape=[
            jax.ShapeDtypeStruct((KP, HD), jnp.float32),
            jax.ShapeDtypeStruct((KP, KP), jnp.float32),
        ],
        scratch_shapes=[pltpu.VMEM((KP, KP), jnp.float32)],
    )(mseg, s1, x1, w2s, w2n, ws2s, ws2n, wh1s, wh1n, wh2s, wh2n)


def kernel(x_note, edge_index, batch, W_embed, b_embed, W1_self, W1_neigh,
           Ws1_self, Ws1_neigh, W2_self, W2_neigh, Ws2_self, Ws2_neigh,
           Wh1_self, Wh1_neigh, Wh2_self, Wh2_neigh):
    del batch
    src = edge_index[0].reshape(NW, NCHUNK, CHUNK)
    dst = edge_index[1].reshape(NW, NCHUNK, CHUNK)
    pad1 = lambda w: jnp.pad(w, ((0, 0), (0, KP - K)))

    h = _tc_embed(x_note, W_embed, b_embed.reshape(1, ED))
    segdeg = _seg_sum_h(h.reshape(N * (OW // (2 * ED)), 2 * ED), src, dst)
    s1p, x1 = _tc_mid(h, segdeg, W1_self, W1_neigh,
                      pad1(Ws1_self), pad1(Ws1_neigh))
    mseg = _seg_sum_s(s1p.reshape(N * (OW // KP), KP), src, dst)
    zout, s2p = _tc_tail(mseg, s1p, x1, W2_self, W2_neigh, pad1(Ws2_self),
                         pad1(Ws2_neigh), Wh1_self, Wh1_neigh, Wh2_self,
                         Wh2_neigh)
    return (zout[:K], s1p[:, :K], s2p[:K, :K])


# speedup vs baseline: 1.2491x; 1.1224x over previous
import functools

import jax
import jax.numpy as jnp
from jax import lax
from jax.experimental import pallas as pl
from jax.experimental.pallas import tpu as pltpu
from jax.experimental.pallas import tpu_sc as plsc

N = 10000
E = 320000
DF = 128
ED = 32
HD = 200
K = 15
KP = 16

NC = 2
NS = 16
NW = NC * NS
EPW = E // NW
CHUNK = 80
NCHUNK = EPW // CHUNK
NBUF = 5
NGROUP = NCHUNK // NBUF
OW = 128
NP = 10240
RPT = NP // NS

def _dot(a, b):
    return jnp.dot(a, b, preferred_element_type=jnp.float32)


def _dot_t(a, b):
    return lax.dot_general(a, b, (((0,), (0,)), ((), ())),
                           preferred_element_type=jnp.float32)


@functools.lru_cache(maxsize=None)
def _make_seg_sum(width, with_deg):
    mesh = plsc.VectorSubcoreMesh(core_axis_name="c", subcore_axis_name="s",
                                  num_cores=NC, num_subcores=NS)
    out_type = [jax.ShapeDtypeStruct((NC, N, OW), jnp.float32)]
    nsem = 3 * NBUF if with_deg else 2 * NBUF
    scratch = [
        pltpu.VMEM((NCHUNK, CHUNK), jnp.int32),
        pltpu.VMEM((NCHUNK, CHUNK), jnp.int32),
        pltpu.VMEM((NBUF, CHUNK, width), jnp.float32),
        pltpu.VMEM((RPT, width), jnp.float32),
        pltpu.VMEM_SHARED((NP, width), jnp.float32),
    ] + [pltpu.SemaphoreType.DMA] * nsem
    if with_deg:
        scratch += [
            pltpu.VMEM((CHUNK, 16), jnp.float32),
            pltpu.VMEM((RPT, 16), jnp.float32),
            pltpu.VMEM_SHARED((NP, 16), jnp.float32),
        ]

    def body(table_hbm, src_hbm, dst_hbm, *refs):
        if with_deg:
            (out_hbm, vsrc, vdst, vrows, vstage, acc) = refs[:6]
            sems = refs[6:6 + nsem]
            vones, vstaged, accd = refs[6 + nsem:]
            gsem, ssem, osem = (sems[:NBUF], sems[NBUF:2 * NBUF],
                                sems[2 * NBUF:])
        else:
            (out_hbm, vsrc, vdst, vrows, vstage, acc) = refs[:6]
            sems = refs[6:6 + nsem]
            gsem, ssem = sems[:NBUF], sems[NBUF:]
        c = lax.axis_index("c")
        s = lax.axis_index("s")
        wid = c * NS + s
        zeros16 = jnp.zeros((16,), jnp.float32)

        def zrow(i, _):
            for j0 in range(width // 16):
                vstage[i, pl.ds(j0 * 16, 16)] = zeros16
            if with_deg:
                vstaged[i, pl.ds(0, 16)] = zeros16
            return 0
        lax.fori_loop(0, RPT, zrow, 0)
        pltpu.sync_copy(vstage, acc.at[pl.ds(s * RPT, RPT)])
        if with_deg:
            pltpu.sync_copy(vstaged, accd.at[pl.ds(s * RPT, RPT)])
            ones16 = jnp.ones((16,), jnp.float32)

            def orow(i, _):
                vones[i, pl.ds(0, 16)] = ones16
                return 0
            lax.fori_loop(0, CHUNK, orow, 0)
        plsc.subcore_barrier()

        pltpu.sync_copy(src_hbm.at[wid], vsrc)
        pltpu.sync_copy(dst_hbm.at[wid], vdst)

        factor = OW // width

        def srow(i, _):
            for k in range(CHUNK // 16):
                vsrc[i, pl.ds(k * 16, 16)] = vsrc[i, pl.ds(k * 16, 16)] * factor
            return 0
        lax.fori_loop(0, NCHUNK, srow, 0)

        def fire_gather(j, b):
            pltpu.async_copy(table_hbm.at[vsrc.at[j]], vrows.at[b], gsem[b])

        def wait_gather(j, b):
            pltpu.make_async_copy(table_hbm.at[vsrc.at[j]], vrows.at[b],
                                  gsem[b]).wait()

        def fire_scatter(j, b):
            pltpu.async_copy(vrows.at[b], acc.at[vdst.at[j]], ssem[b],
                             add=True)
            if with_deg:
                pltpu.async_copy(vones, accd.at[vdst.at[j]], osem[b],
                                 add=True)

        def wait_scatter(j, b):
            pltpu.make_async_copy(vrows.at[b], acc.at[vdst.at[j]],
                                  ssem[b]).wait()
            if with_deg:
                pltpu.make_async_copy(vones, accd.at[vdst.at[j]],
                                      osem[b]).wait()

        for b in range(NBUF):
            fire_gather(b, b)

        def group(gi, _):
            j0 = gi * NBUF
            for b in range(NBUF):
                wait_gather(j0 + b, b)
                fire_scatter(j0 + b, b)
            for b in range(NBUF):
                wait_scatter(j0 + b, b)
                fire_gather(j0 + NBUF + b, b)
            return 0
        lax.fori_loop(0, NGROUP - 1, group, 0)

        j0 = (NGROUP - 1) * NBUF
        for b in range(NBUF):
            wait_gather(j0 + b, b)
            fire_scatter(j0 + b, b)
        for b in range(NBUF):
            wait_scatter(j0 + b, b)
        plsc.subcore_barrier()

        def writeout(nrows):
            base = s * RPT
            pltpu.sync_copy(acc.at[pl.ds(base, nrows)],
                            vstage.at[pl.ds(0, nrows)])
            if with_deg:
                pltpu.sync_copy(
                    vstage.at[pl.ds(0, nrows)],
                    out_hbm.at[c, pl.ds(base, nrows), pl.ds(0, width)])
                pltpu.sync_copy(accd.at[pl.ds(base, nrows)],
                                vstaged.at[pl.ds(0, nrows)])
                pltpu.sync_copy(
                    vstaged.at[pl.ds(0, nrows)],
                    out_hbm.at[c, pl.ds(base, nrows), pl.ds(width, 16)])
            else:
                pltpu.sync_copy(
                    vstage.at[pl.ds(0, nrows)],
                    out_hbm.at[c, pl.ds(base, nrows), pl.ds(0, width)])

        last = N - (NS - 1) * RPT

        @pl.when(s < NS - 1)
        def _():
            writeout(RPT)

        @pl.when(s == NS - 1)
        def _():
            writeout(last)

    return pl.kernel(body, out_type=out_type, mesh=mesh,
                     scratch_types=scratch,
                     compiler_params=pltpu.CompilerParams(
                         use_tc_tiling_on_sc=False))


def _seg_sum_h(table, src, dst):
    (segdeg,) = _make_seg_sum(ED, True)(table, src, dst)
    return segdeg


def _seg_sum_s(table, src, dst):
    (mseg,) = _make_seg_sum(KP, False)(table, src, dst)
    return mseg


_GRID = 5
_ROWS = N // _GRID


def _embed_body(x_ref, w_ref, b_ref, h_ref):
    h_ref[:, :ED] = _dot(x_ref[...], w_ref[...]) + b_ref[...]
    h_ref[:, ED:] = jnp.zeros((h_ref.shape[0], OW - ED), jnp.float32)


@jax.jit
def _tc_embed(x, w, b):
    return pl.pallas_call(
        _embed_body,
        grid=(_GRID,),
        in_specs=[
            pl.BlockSpec((_ROWS, DF), lambda i: (i, 0)),
            pl.BlockSpec((DF, ED), lambda i: (0, 0)),
            pl.BlockSpec((1, ED), lambda i: (0, 0)),
        ],
        out_specs=pl.BlockSpec((_ROWS, OW), lambda i: (i, 0)),
        out_shape=jax.ShapeDtypeStruct((N, OW), jnp.float32),
    )(x, w, b)


def _mid_body(h_ref, sd_ref, w1s_ref, w1n_ref, ws1s_ref, ws1n_ref,
              s1_ref, x1_ref, x1_acc):
    i = pl.program_id(0)
    h = h_ref[:, :ED]
    seg = sd_ref[0, :, :ED] + sd_ref[1, :, :ED]
    deg = sd_ref[0, :, ED:ED + 1] + sd_ref[1, :, ED:ED + 1]
    agg = seg / jnp.maximum(deg, 1.0)
    z1 = jnp.maximum(_dot(h, w1s_ref[...]) + _dot(agg, w1n_ref[...]), 0.0)
    logits = _dot(h, ws1s_ref[...]) + _dot(agg, ws1n_ref[...])
    col = lax.broadcasted_iota(jnp.int32, logits.shape, 1)
    logits = jnp.where(col >= K, -1e30, logits)
    m = jnp.max(logits, axis=1, keepdims=True)
    e = jnp.exp(logits - m)
    s1 = e / jnp.sum(e, axis=1, keepdims=True)
    s1_ref[:, :KP] = s1
    s1_ref[:, KP:] = jnp.zeros((s1.shape[0], OW - KP), jnp.float32)

    @pl.when(i == 0)
    def _():
        x1_acc[...] = jnp.zeros_like(x1_acc)

    x1_acc[...] += _dot_t(s1, z1)

    @pl.when(i == _GRID - 1)
    def _():
        x1_ref[...] = x1_acc[...]


@jax.jit
def _tc_mid(h, segdeg, w1s, w1n, ws1s, ws1n):
    return pl.pallas_call(
        _mid_body,
        grid=(_GRID,),
        in_specs=[
            pl.BlockSpec((_ROWS, OW), lambda i: (i, 0)),
            pl.BlockSpec((NC, _ROWS, OW), lambda i: (0, i, 0)),
            pl.BlockSpec((ED, ED), lambda i: (0, 0)),
            pl.BlockSpec((ED, ED), lambda i: (0, 0)),
            pl.BlockSpec((ED, KP), lambda i: (0, 0)),
            pl.BlockSpec((ED, KP), lambda i: (0, 0)),
        ],
        out_specs=[
            pl.BlockSpec((_ROWS, OW), lambda i: (i, 0)),
            pl.BlockSpec((KP, ED), lambda i: (0, 0)),
        ],
        out_shape=[
            jax.ShapeDtypeStruct((N, OW), jnp.float32),
            jax.ShapeDtypeStruct((KP, ED), jnp.float32),
        ],
        scratch_shapes=[pltpu.VMEM((KP, ED), jnp.float32)],
    )(h, segdeg, w1s, w1n, ws1s, ws1n)


def _tail_body(m_ref, s1_ref, x1_ref, w2s_ref, w2n_ref, ws2s_ref, ws2n_ref,
               wh1s_ref, wh1n_ref, wh2s_ref, wh2n_ref,
               zout_ref, s2_ref, a1_acc):
    i = pl.program_id(0)

    @pl.when(i == 0)
    def _():
        a1_acc[...] = jnp.zeros_like(a1_acc)

    m = m_ref[0, :, :KP] + m_ref[1, :, :KP]
    a1_acc[...] += _dot_t(m, s1_ref[:, :KP])

    @pl.when(i == _GRID - 1)
    def _():
        a1 = a1_acc[...]
        x1 = x1_ref[...]
        d1 = jnp.maximum(jnp.sum(a1, axis=1, keepdims=True), 1.0)
        agg1 = _dot(a1, x1) / d1
        z2 = jnp.maximum(_dot(x1, w2s_ref[...]) + _dot(agg1, w2n_ref[...]),
                         0.0)
        l2 = _dot(x1, ws2s_ref[...]) + _dot(agg1, ws2n_ref[...])
        col = lax.broadcasted_iota(jnp.int32, l2.shape, 1)
        l2 = jnp.where(col >= K, -1e30, l2)
        mx = jnp.max(l2, axis=1, keepdims=True)
        ex = jnp.exp(l2 - mx)
        s2 = ex / jnp.sum(ex, axis=1, keepdims=True)
        s2_ref[...] = s2
        x2 = _dot_t(s2, z2)
        a2 = _dot(_dot_t(s2, a1), s2)
        d2 = jnp.maximum(jnp.sum(a2, axis=1, keepdims=True), 1.0)
        h1 = jnp.maximum(
            _dot(x2, wh1s_ref[...]) + _dot(_dot(a2, x2) / d2, wh1n_ref[...]),
            0.0)
        zout_ref[...] = jnp.maximum(
            _dot(h1, wh2s_ref[...]) + _dot(_dot(a2, h1) / d2, wh2n_ref[...]),
            0.0)


@jax.jit
def _tc_tail(mseg, s1, x1, w2s, w2n, ws2s, ws2n, wh1s, wh1n, wh2s, wh2n):
    small = lambda r, c: pl.BlockSpec((r, c), lambda i: (0, 0))
    return pl.pallas_call(
        _tail_body,
        grid=(_GRID,),
        in_specs=[
            pl.BlockSpec((NC, _ROWS, OW), lambda i: (0, i, 0)),
            pl.BlockSpec((_ROWS, OW), lambda i: (i, 0)),
            small(KP, ED),
            small(ED, ED), small(ED, ED), small(ED, KP), small(ED, KP),
            small(ED, HD), small(ED, HD), small(HD, HD), small(HD, HD),
        ],
        out_specs=[
            pl.BlockSpec((KP, HD), lambda i: (0, 0)),
            pl.BlockSpec((KP, KP), lambda i: (0, 0)),
        ],
        out_shape=[
            jax.ShapeDtypeStruct((KP, HD), jnp.float32),
            jax.ShapeDtypeStruct((KP, KP), jnp.float32),
        ],
        scratch_shapes=[pltpu.VMEM((KP, KP), jnp.float32)],
    )(mseg, s1, x1, w2s, w2n, ws2s, ws2n, wh1s, wh1n, wh2s, wh2n)


def kernel(x_note, edge_index, batch, W_embed, b_embed, W1_self, W1_neigh,
           Ws1_self, Ws1_neigh, W2_self, W2_neigh, Ws2_self, Ws2_neigh,
           Wh1_self, Wh1_neigh, Wh2_self, Wh2_neigh):
    del batch
    src = edge_index[0].reshape(NW, NCHUNK, CHUNK)
    dst = edge_index[1].reshape(NW, NCHUNK, CHUNK)
    pad1 = lambda w: jnp.pad(w, ((0, 0), (0, KP - K)))

    h = _tc_embed(x_note, W_embed, b_embed.reshape(1, ED))
    segdeg = _seg_sum_h(h.reshape(N * (OW // ED), ED), src, dst)
    s1p, x1 = _tc_mid(h, segdeg, W1_self, W1_neigh,
                      pad1(Ws1_self), pad1(Ws1_neigh))
    mseg = _seg_sum_s(s1p.reshape(N * (OW // KP), KP), src, dst)
    zout, s2p = _tc_tail(mseg, s1p, x1, W2_self, W2_neigh, pad1(Ws2_self),
                         pad1(Ws2_neigh), Wh1_self, Wh1_neigh, Wh2_self,
                         Wh2_neigh)
    return (zout[:K], s1p[:, :K], s2p[:K, :K])
